# baseline (speedup 1.0000x reference)
import functools

import jax
import jax.numpy as jnp
from jax import lax
from jax.experimental import pallas as pl
from jax.experimental.pallas import tpu as pltpu
from jax.experimental.pallas import tpu_sc as plsc

N = 100000
D_FEAT = 128
EMB = 128
NF_EMB = 64
B = 16384
OUT_D = EMB + NF_EMB

NUM_CORES = 2
NUM_SUBCORES = 16
NUM_WORKERS = NUM_CORES * NUM_SUBCORES

NCHUNK = 2
CB = B // NCHUNK
BPW = CB // NUM_WORKERS
BN = 4096


def _sc_gather_chunk(emb_weight, node_feats, idx_chunk):
    mesh = plsc.VectorSubcoreMesh(core_axis_name="c", subcore_axis_name="s")

    @functools.partial(
        pl.kernel,
        mesh=mesh,
        out_type=(
            jax.ShapeDtypeStruct((CB, EMB), jnp.float32),
            jax.ShapeDtypeStruct((CB, D_FEAT), jnp.float32),
        ),
        scratch_types=[
            pltpu.VMEM((BPW,), jnp.int32),
            pltpu.VMEM((BPW, EMB), jnp.float32),
            pltpu.SemaphoreType.DMA,
        ],
    )
    def k(emb_hbm, nf_hbm, idx_hbm, emb_out, nf_out, idx_v, rows_v, sem):
        wid = lax.axis_index("s") * NUM_CORES + lax.axis_index("c")
        base = wid * BPW
        pltpu.sync_copy(idx_hbm.at[pl.ds(base, BPW)], idx_v)
        pltpu.async_copy(emb_hbm.at[idx_v], rows_v, sem).wait()
        pltpu.sync_copy(rows_v, emb_out.at[pl.ds(base, BPW)])
        pltpu.async_copy(nf_hbm.at[idx_v], rows_v, sem).wait()
        pltpu.sync_copy(rows_v, nf_out.at[pl.ds(base, BPW)])

    return k(emb_weight, node_feats, idx_chunk)


def _tc_fuse_kernel(emb_ref, nf_ref, w_ref, b_ref, out_ref):
    zT = jax.lax.dot_general(
        w_ref[...], nf_ref[...],
        dimension_numbers=(((0,), (1,)), ((), ())),
        preferred_element_type=jnp.float32,
    )
    out_ref[:EMB, :] = emb_ref[...].T
    out_ref[EMB:, :] = zT + b_ref[...]


def _tc_fuse_first(emb_rows, nf_rows, lin_W, lin_b):
    return pl.pallas_call(
        _tc_fuse_kernel,
        grid=(CB // BN,),
        in_specs=[
            pl.BlockSpec((BN, EMB), lambda i: (i, 0)),
            pl.BlockSpec((BN, D_FEAT), lambda i: (i, 0)),
            pl.BlockSpec((D_FEAT, NF_EMB), lambda i: (0, 0)),
            pl.BlockSpec((NF_EMB, 1), lambda i: (0, 0)),
        ],
        out_specs=pl.BlockSpec((OUT_D, BN), lambda i: (0, i)),
        out_shape=jax.ShapeDtypeStruct((OUT_D, B), jnp.float32),
    )(emb_rows, nf_rows, lin_W, lin_b)


def _tc_fuse_second_kernel(acc_ref, emb_ref, nf_ref, w_ref, b_ref, out_ref):
    del acc_ref
    _tc_fuse_kernel(emb_ref, nf_ref, w_ref, b_ref, out_ref)


def _tc_fuse_second(acc, emb_rows, nf_rows, lin_W, lin_b):
    nblk = CB // BN
    return pl.pallas_call(
        _tc_fuse_second_kernel,
        grid=(nblk,),
        in_specs=[
            pl.BlockSpec((8, 128), lambda i: (0, 0)),
            pl.BlockSpec((BN, EMB), lambda i: (i, 0)),
            pl.BlockSpec((BN, D_FEAT), lambda i: (i, 0)),
            pl.BlockSpec((D_FEAT, NF_EMB), lambda i: (0, 0)),
            pl.BlockSpec((NF_EMB, 1), lambda i: (0, 0)),
        ],
        out_specs=pl.BlockSpec((OUT_D, BN), lambda i: (0, i + nblk)),
        out_shape=jax.ShapeDtypeStruct((OUT_D, B), jnp.float32),
        input_output_aliases={0: 0},
    )(acc, emb_rows, nf_rows, lin_W, lin_b)


def kernel(node_feats, emb_weight, lin_W, lin_b, batch):
    b2 = lin_b.reshape(NF_EMB, 1)
    emb0, nf0 = _sc_gather_chunk(emb_weight, node_feats, batch[:CB])
    emb1, nf1 = _sc_gather_chunk(emb_weight, node_feats, batch[CB:])
    acc = _tc_fuse_first(emb0, nf0, lin_W, b2)
    out = _tc_fuse_second(acc, emb1, nf1, lin_W, b2)
    return out.T


# speedup vs baseline: 1.1034x; 1.1034x over previous
import functools

import jax
import jax.numpy as jnp
from jax import lax
from jax.experimental import pallas as pl
from jax.experimental.pallas import tpu as pltpu
from jax.experimental.pallas import tpu_sc as plsc

N = 100000
D_FEAT = 128
EMB = 128
NF_EMB = 64
B = 16384
OUT_D = EMB + NF_EMB

NUM_CORES = 2
NUM_SUBCORES = 16
NUM_WORKERS = NUM_CORES * NUM_SUBCORES
B_PER_W = B // NUM_WORKERS

BN = 8192


def _sc_gather2(emb_weight, node_feats, batch):
    mesh = plsc.VectorSubcoreMesh(core_axis_name="c", subcore_axis_name="s")

    @functools.partial(
        pl.kernel,
        mesh=mesh,
        out_type=(
            jax.ShapeDtypeStruct((B, EMB), jnp.float32),
            jax.ShapeDtypeStruct((B, D_FEAT), jnp.float32),
        ),
        scratch_types=[
            pltpu.VMEM((B_PER_W,), jnp.int32),
            pltpu.VMEM((B_PER_W, EMB), jnp.float32),
            pltpu.SemaphoreType.DMA,
        ],
    )
    def k(emb_hbm, nf_hbm, idx_hbm, emb_out, nf_out, idx_v, rows_v, sem):
        wid = lax.axis_index("s") * NUM_CORES + lax.axis_index("c")
        base = wid * B_PER_W
        pltpu.sync_copy(idx_hbm.at[pl.ds(base, B_PER_W)], idx_v)
        pltpu.async_copy(emb_hbm.at[idx_v], rows_v, sem).wait()
        pltpu.sync_copy(rows_v, emb_out.at[pl.ds(base, B_PER_W)])
        pltpu.async_copy(nf_hbm.at[idx_v], rows_v, sem).wait()
        pltpu.sync_copy(rows_v, nf_out.at[pl.ds(base, B_PER_W)])

    return k(emb_weight, node_feats, batch)


def _tc_fuse_kernel(emb_ref, nf_ref, w_ref, b_ref, out_ref):
    zT = jax.lax.dot_general(
        w_ref[...], nf_ref[...],
        dimension_numbers=(((0,), (1,)), ((), ())),
        preferred_element_type=jnp.float32,
    )
    out_ref[:EMB, :] = emb_ref[...].T
    out_ref[EMB:, :] = zT + b_ref[...]


def _tc_fuse(emb_rows, nf_rows, lin_W, lin_b):
    return pl.pallas_call(
        _tc_fuse_kernel,
        grid=(B // BN,),
        in_specs=[
            pl.BlockSpec((BN, EMB), lambda i: (i, 0)),
            pl.BlockSpec((BN, D_FEAT), lambda i: (i, 0)),
            pl.BlockSpec((D_FEAT, NF_EMB), lambda i: (0, 0)),
            pl.BlockSpec((NF_EMB, 1), lambda i: (0, 0)),
        ],
        out_specs=pl.BlockSpec((OUT_D, BN), lambda i: (0, i)),
        out_shape=jax.ShapeDtypeStruct((OUT_D, B), jnp.float32),
    )(emb_rows, nf_rows, lin_W, lin_b)


def kernel(node_feats, emb_weight, lin_W, lin_b, batch):
    emb_rows, nf_rows = _sc_gather2(emb_weight, node_feats, batch)
    return _tc_fuse(emb_rows, nf_rows, lin_W, lin_b.reshape(NF_EMB, 1)).T
